# baseline (speedup 1.0000x reference)
import jax
import jax.numpy as jnp
from jax import lax
from jax.experimental import pallas as pl
from jax.experimental.pallas import tpu as pltpu

GS = 8
TOPK = 3


def _pool_body(xw_ref, d_ref):
    d_ref[...] = jnp.max(xw_ref[...], axis=1)


def _topk_body(d_rows_ref, d_all_ref, val_ref, idx_ref):
    s = lax.dot_general(d_rows_ref[...], d_all_ref[...],
                        (((1,), (1,)), ((), ())),
                        preferred_element_type=jnp.float32)
    iota = lax.broadcasted_iota(jnp.int32, s.shape, 1)
    vals, idxs = [], []
    for _ in range(TOPK):
        v = jnp.max(s, axis=1, keepdims=True)
        i = jnp.min(jnp.where(s >= v, iota, jnp.int32(2 ** 30)),
                    axis=1, keepdims=True)
        vals.append(v)
        idxs.append(i)
        s = jnp.where(iota == i, -jnp.inf, s)
    r = s.shape[0]
    val_ref[...] = jnp.concatenate(
        vals + [jnp.full((r, 8 - TOPK), -1e30, jnp.float32)], axis=1)
    idx_ref[...] = jnp.concatenate(
        idxs + [jnp.zeros((r, 8 - TOPK), jnp.int32)], axis=1)


WB = 8


def _attn_body(*refs):
    (sidx_ref, xs_ref), nrefs = refs[:2], refs[2:2 + TOPK * WB]
    (sv_ref, wq_ref, wk_ref, wv_ref, wp_ref,
     bq_ref, bk_ref, bv_ref, bp_ref, lt_ref, lb_ref, out_ref) \
        = refs[2 + TOPK * WB:]
    p_dim = GS * GS
    c_dim = xs_ref.shape[-1]
    rows = WB * p_dim

    xs = xs_ref[...].reshape(rows, c_dim)

    sv = sv_ref[...].reshape(WB, 8)
    lane = lax.broadcasted_iota(jnp.int32, sv.shape, 1)
    m3 = jnp.max(jnp.where(lane < TOPK, sv, -jnp.inf), axis=1, keepdims=True)
    e = jnp.where(lane < TOPK, jnp.exp(sv - m3), 0.0)
    w = e / jnp.sum(e, axis=1, keepdims=True)

    ctx = jnp.concatenate([
        sum(w[j:j + 1, k:k + 1] * nrefs[TOPK * j + k][0] for k in range(TOPK))
        for j in range(WB)], axis=0)

    def mm(a, b, dims):
        return lax.dot_general(a.astype(jnp.bfloat16), b.astype(jnp.bfloat16),
                               (dims, ((), ())),
                               preferred_element_type=jnp.float32)

    q = mm(xs, wq_ref[...], ((1,), (1,))) + bq_ref[...]
    kk = mm(ctx, wk_ref[...], ((1,), (1,))) + bk_ref[...]
    vv = mm(ctx, wv_ref[...], ((1,), (1,))) + bv_ref[...]

    pvec = lax.broadcasted_iota(jnp.int32, (rows, 1), 0)
    yy = (pvec % p_dim) // GS
    xx = pvec % GS
    acc = jnp.zeros((rows, c_dim), jnp.float32)
    for ky in range(3):
        for kx in range(3):
            dy, dx = ky - 1, kx - 1
            s = GS * dy + dx
            sh = xs if s == 0 else jnp.roll(xs, -s, axis=0)
            valid = ((yy + dy >= 0) & (yy + dy < GS)
                     & (xx + dx >= 0) & (xx + dx < GS))
            t = 3 * ky + kx
            acc = acc + jnp.where(valid, sh, 0.0) * lt_ref[t:t + 1, :]
    lepe = acc + lb_ref[...]

    scale = c_dim ** -0.5
    outs = []
    for j in range(WB):
        sl = slice(j * p_dim, (j + 1) * p_dim)
        attn = mm(q[sl], kk[sl], ((1,), (1,))) * scale
        mrow = jnp.max(attn, axis=1, keepdims=True)
        pexp = jnp.exp(attn - mrow)
        pnorm = pexp / jnp.sum(pexp, axis=1, keepdims=True)
        outs.append(mm(pnorm, vv[sl], ((1,), (0,))))
    o = jnp.concatenate(outs, axis=0) + lepe
    res = mm(o, wp_ref[...], ((1,), (1,))) + bp_ref[...]
    out_ref[...] = res.reshape(WB, p_dim, c_dim)


def kernel(x, Wq, bq, Wk, bk, Wv, bv, Wp, bp, lepe_w, lepe_b):
    b, c, h, w = x.shape
    gh, gw = h // GS, w // GS
    n = b * gh * gw
    p = GS * GS

    xw = (jnp.transpose(x, (0, 2, 3, 1))
          .reshape(b * gh, GS, gw, GS, c)
          .transpose(0, 2, 1, 3, 4)
          .reshape(n, p, c))

    pool_blk = min(128, n)
    d = pl.pallas_call(
        _pool_body,
        grid=(n // pool_blk,),
        in_specs=[pl.BlockSpec((pool_blk, p, c), lambda i: (i, 0, 0))],
        out_specs=pl.BlockSpec((pool_blk, c), lambda i: (i, 0)),
        out_shape=jax.ShapeDtypeStruct((n, c), jnp.float32),
    )(xw)

    r = min(256, n)
    sval, sidx = pl.pallas_call(
        _topk_body,
        grid=(n // r,),
        in_specs=[pl.BlockSpec((r, c), lambda i: (i, 0)),
                  pl.BlockSpec((n, c), lambda i: (0, 0))],
        out_specs=[pl.BlockSpec((r, 8), lambda i: (i, 0)),
                   pl.BlockSpec((r, 8), lambda i: (i, 0))],
        out_shape=[jax.ShapeDtypeStruct((n, 8), jnp.float32),
                   jax.ShapeDtypeStruct((n, 8), jnp.int32)],
    )(d, d)

    sidx_flat = sidx[:, :TOPK].reshape(-1)
    sval3 = sval.reshape(n, 1, 8)
    row = lambda v: v.reshape(1, c)
    lt = jnp.pad(lepe_w.reshape(c, 9).transpose(1, 0), ((0, 7), (0, 0)))

    wblk = lambda: pl.BlockSpec((c, c), lambda i, si: (0, 0))
    bblk = lambda: pl.BlockSpec((1, c), lambda i, si: (0, 0))
    nspec = lambda t: pl.BlockSpec(
        (1, p, c), lambda i, si, t=t: (si[TOPK * WB * i + t], 0, 0))
    grid_spec = pltpu.PrefetchScalarGridSpec(
        num_scalar_prefetch=1,
        grid=(n // WB,),
        in_specs=[
            pl.BlockSpec((WB, p, c), lambda i, si: (i, 0, 0)),
            *[nspec(t) for t in range(TOPK * WB)],
            pl.BlockSpec((WB, 1, 8), lambda i, si: (i, 0, 0)),
            wblk(), wblk(), wblk(), wblk(),
            bblk(), bblk(), bblk(), bblk(),
            pl.BlockSpec((16, c), lambda i, si: (0, 0)),
            bblk(),
        ],
        out_specs=pl.BlockSpec((WB, p, c), lambda i, si: (i, 0, 0)),
    )
    out = pl.pallas_call(
        _attn_body,
        grid_spec=grid_spec,
        out_shape=jax.ShapeDtypeStruct((n, p, c), jnp.float32),
    )(sidx_flat, xw, *([xw] * (TOPK * WB)), sval3, Wq, Wk, Wv, Wp,
      row(bq), row(bk), row(bv), row(bp), lt, row(lepe_b))

    out = (out.reshape(b, gh, gw, GS, GS, c)
           .transpose(0, 5, 1, 3, 2, 4)
           .reshape(b, c, h, w))
    return out


# speedup vs baseline: 1.4641x; 1.4641x over previous
import jax
import jax.numpy as jnp
from jax import lax
from jax.experimental import pallas as pl
from jax.experimental.pallas import tpu as pltpu

GS = 8
TOPK = 3


def _pool_body(xw_ref, d_ref):
    d_ref[...] = jnp.max(xw_ref[...], axis=1)


def _topk_body(d_rows_ref, d_all_ref, val_ref, idx_ref):
    s = lax.dot_general(d_rows_ref[...], d_all_ref[...],
                        (((1,), (1,)), ((), ())),
                        preferred_element_type=jnp.float32)
    iota = lax.broadcasted_iota(jnp.int32, s.shape, 1)
    vals, idxs = [], []
    for _ in range(TOPK):
        v = jnp.max(s, axis=1, keepdims=True)
        i = jnp.min(jnp.where(s >= v, iota, jnp.int32(2 ** 30)),
                    axis=1, keepdims=True)
        vals.append(v)
        idxs.append(i)
        s = jnp.where(iota == i, -jnp.inf, s)
    r = s.shape[0]
    val_ref[...] = jnp.concatenate(
        vals + [jnp.full((r, 8 - TOPK), -1e30, jnp.float32)], axis=1)
    idx_ref[...] = jnp.concatenate(
        idxs + [jnp.zeros((r, 8 - TOPK), jnp.int32)], axis=1)


WB = 8


def _attn_body(*refs):
    (sidx_ref, xs_ref), nrefs = refs[:2], refs[2:2 + TOPK * WB]
    sv_ref, wcat_ref, misc_ref, out_ref = refs[2 + TOPK * WB:]
    p_dim = GS * GS
    c_dim = xs_ref.shape[-1]
    rows = WB * p_dim

    xs = xs_ref[...].reshape(rows, c_dim)

    sv = sv_ref[...].reshape(WB, 8)
    lane = lax.broadcasted_iota(jnp.int32, sv.shape, 1)
    m3 = jnp.max(jnp.where(lane < TOPK, sv, -jnp.inf), axis=1, keepdims=True)
    e = jnp.where(lane < TOPK, jnp.exp(sv - m3), 0.0)
    w = e / jnp.sum(e, axis=1, keepdims=True)

    ctx = jnp.concatenate([
        sum(w[j:j + 1, k:k + 1] * nrefs[TOPK * j + k][0] for k in range(TOPK))
        for j in range(WB)], axis=0)

    def mm(a, b, dims):
        return lax.dot_general(a, b, (dims, ((), ())),
                               preferred_element_type=jnp.float32)

    wcat = wcat_ref[...]
    misc = misc_ref[...]
    q = mm(xs, wcat[0:c_dim], ((1,), (1,))) + misc[0:1]
    kk = mm(ctx, wcat[c_dim:2 * c_dim], ((1,), (1,))) + misc[1:2]
    vv = mm(ctx, wcat[2 * c_dim:3 * c_dim], ((1,), (1,))) + misc[2:3]

    pvec = lax.broadcasted_iota(jnp.int32, (rows, 1), 0)
    yy = (pvec % p_dim) // GS
    xx = pvec % GS
    acc = jnp.zeros((rows, c_dim), jnp.float32)
    for ky in range(3):
        for kx in range(3):
            dy, dx = ky - 1, kx - 1
            s = GS * dy + dx
            sh = xs if s == 0 else jnp.roll(xs, -s, axis=0)
            valid = ((yy + dy >= 0) & (yy + dy < GS)
                     & (xx + dx >= 0) & (xx + dx < GS))
            t = 3 * ky + kx
            acc = acc + jnp.where(valid, sh, 0.0) * misc[4 + t:5 + t, :]
    lepe = acc + misc[13:14]

    scale = c_dim ** -0.5
    attn = jnp.concatenate(
        [mm(q[j * p_dim:(j + 1) * p_dim], kk[j * p_dim:(j + 1) * p_dim],
            ((1,), (1,))) for j in range(WB)], axis=0) * scale
    mrow = jnp.max(attn, axis=1, keepdims=True)
    pexp = jnp.exp(attn - mrow)
    pnorm = pexp / jnp.sum(pexp, axis=1, keepdims=True)
    o = jnp.concatenate(
        [mm(pnorm[j * p_dim:(j + 1) * p_dim], vv[j * p_dim:(j + 1) * p_dim],
            ((1,), (0,))) for j in range(WB)], axis=0) + lepe
    res = mm(o, wcat[3 * c_dim:4 * c_dim], ((1,), (1,))) + misc[3:4]
    out_ref[...] = res.reshape(WB, p_dim, c_dim)


def kernel(x, Wq, bq, Wk, bk, Wv, bv, Wp, bp, lepe_w, lepe_b):
    b, c, h, w = x.shape
    gh, gw = h // GS, w // GS
    n = b * gh * gw
    p = GS * GS

    xw = (jnp.transpose(x, (0, 2, 3, 1))
          .reshape(b * gh, GS, gw, GS, c)
          .transpose(0, 2, 1, 3, 4)
          .reshape(n, p, c))

    pool_blk = min(128, n)
    d = pl.pallas_call(
        _pool_body,
        grid=(n // pool_blk,),
        in_specs=[pl.BlockSpec((pool_blk, p, c), lambda i: (i, 0, 0))],
        out_specs=pl.BlockSpec((pool_blk, c), lambda i: (i, 0)),
        out_shape=jax.ShapeDtypeStruct((n, c), jnp.float32),
    )(xw)

    r = min(256, n)
    sval, sidx = pl.pallas_call(
        _topk_body,
        grid=(n // r,),
        in_specs=[pl.BlockSpec((r, c), lambda i: (i, 0)),
                  pl.BlockSpec((n, c), lambda i: (0, 0))],
        out_specs=[pl.BlockSpec((r, 8), lambda i: (i, 0)),
                   pl.BlockSpec((r, 8), lambda i: (i, 0))],
        out_shape=[jax.ShapeDtypeStruct((n, 8), jnp.float32),
                   jax.ShapeDtypeStruct((n, 8), jnp.int32)],
    )(d, d)

    sidx_flat = sidx[:, :TOPK].reshape(-1)
    sval3 = sval.reshape(n, 1, 8)
    lt = lepe_w.reshape(c, 9).transpose(1, 0)
    wcat = jnp.concatenate([Wq, Wk, Wv, Wp], axis=0)
    misc = jnp.concatenate(
        [bq.reshape(1, c), bk.reshape(1, c), bv.reshape(1, c),
         bp.reshape(1, c), lt, lepe_b.reshape(1, c),
         jnp.zeros((2, c), jnp.float32)], axis=0)

    nspec = lambda t: pl.BlockSpec(
        (1, p, c), lambda i, si, t=t: (si[TOPK * WB * i + t], 0, 0))
    grid_spec = pltpu.PrefetchScalarGridSpec(
        num_scalar_prefetch=1,
        grid=(n // WB,),
        in_specs=[
            pl.BlockSpec((WB, p, c), lambda i, si: (i, 0, 0)),
            *[nspec(t) for t in range(TOPK * WB)],
            pl.BlockSpec((WB, 1, 8), lambda i, si: (i, 0, 0)),
            pl.BlockSpec((4 * c, c), lambda i, si: (0, 0)),
            pl.BlockSpec((16, c), lambda i, si: (0, 0)),
        ],
        out_specs=pl.BlockSpec((WB, p, c), lambda i, si: (i, 0, 0)),
    )
    out = pl.pallas_call(
        _attn_body,
        grid_spec=grid_spec,
        out_shape=jax.ShapeDtypeStruct((n, p, c), jnp.float32),
    )(sidx_flat, xw, *([xw] * (TOPK * WB)), sval3, wcat, misc)

    out = (out.reshape(b, gh, gw, GS, GS, c)
           .transpose(0, 5, 1, 3, 2, 4)
           .reshape(b, c, h, w))
    return out


# speedup vs baseline: 1.7003x; 1.1613x over previous
import jax
import jax.numpy as jnp
from jax import lax
from jax.experimental import pallas as pl
from jax.experimental.pallas import tpu as pltpu

GS = 8
TOPK = 3


def _pool_body(xw_ref, d_ref):
    d_ref[...] = jnp.max(xw_ref[...], axis=1)


def _topk_body(d_rows_ref, d_all_ref, val_ref, idx_ref):
    s = lax.dot_general(d_rows_ref[...], d_all_ref[...],
                        (((1,), (1,)), ((), ())),
                        preferred_element_type=jnp.float32)
    iota = lax.broadcasted_iota(jnp.int32, s.shape, 1)
    vals, idxs = [], []
    for _ in range(TOPK):
        v = jnp.max(s, axis=1, keepdims=True)
        i = jnp.min(jnp.where(s >= v, iota, jnp.int32(2 ** 30)),
                    axis=1, keepdims=True)
        vals.append(v)
        idxs.append(i)
        s = jnp.where(iota == i, -jnp.inf, s)
    r = s.shape[0]
    val_ref[...] = jnp.concatenate(
        vals + [jnp.full((r, 8 - TOPK), -1e30, jnp.float32)], axis=1)
    idx_ref[...] = jnp.concatenate(
        idxs + [jnp.zeros((r, 8 - TOPK), jnp.int32)], axis=1)


WB = 16


def _attn_body(*refs):
    (sidx_ref, xs_ref), nrefs = refs[:2], refs[2:2 + TOPK * WB]
    sv_ref, wcat_ref, misc_ref, out_ref = refs[2 + TOPK * WB:]
    p_dim = GS * GS
    c_dim = xs_ref.shape[-1]
    rows = WB * p_dim

    xs = xs_ref[...].reshape(rows, c_dim)

    sv = sv_ref[...].reshape(WB, 8)
    lane = lax.broadcasted_iota(jnp.int32, sv.shape, 1)
    m3 = jnp.max(jnp.where(lane < TOPK, sv, -jnp.inf), axis=1, keepdims=True)
    e = jnp.where(lane < TOPK, jnp.exp(sv - m3), 0.0)
    w = e / jnp.sum(e, axis=1, keepdims=True)

    ctx = jnp.concatenate([
        sum(w[j:j + 1, k:k + 1] * nrefs[TOPK * j + k][0] for k in range(TOPK))
        for j in range(WB)], axis=0)

    def mm(a, b, dims):
        return lax.dot_general(a, b, (dims, ((), ())),
                               preferred_element_type=jnp.float32)

    wcat = wcat_ref[...]
    misc = misc_ref[...]
    q = mm(xs, wcat[0:c_dim], ((1,), (1,))) + misc[0:1]
    kk = mm(ctx, wcat[c_dim:2 * c_dim], ((1,), (1,))) + misc[1:2]
    vv = mm(ctx, wcat[2 * c_dim:3 * c_dim], ((1,), (1,))) + misc[2:3]

    pvec = lax.broadcasted_iota(jnp.int32, (rows, 1), 0)
    yy = (pvec % p_dim) // GS
    xx = pvec % GS
    acc = jnp.zeros((rows, c_dim), jnp.float32)
    for ky in range(3):
        for kx in range(3):
            dy, dx = ky - 1, kx - 1
            s = GS * dy + dx
            sh = xs if s == 0 else jnp.roll(xs, -s, axis=0)
            valid = ((yy + dy >= 0) & (yy + dy < GS)
                     & (xx + dx >= 0) & (xx + dx < GS))
            t = 3 * ky + kx
            acc = acc + jnp.where(valid, sh, 0.0) * misc[4 + t:5 + t, :]
    lepe = acc + misc[13:14]

    scale = c_dim ** -0.5
    attn = jnp.concatenate(
        [mm(q[j * p_dim:(j + 1) * p_dim], kk[j * p_dim:(j + 1) * p_dim],
            ((1,), (1,))) for j in range(WB)], axis=0) * scale
    mrow = jnp.max(attn, axis=1, keepdims=True)
    pexp = jnp.exp(attn - mrow)
    pnorm = pexp / jnp.sum(pexp, axis=1, keepdims=True)
    o = jnp.concatenate(
        [mm(pnorm[j * p_dim:(j + 1) * p_dim], vv[j * p_dim:(j + 1) * p_dim],
            ((1,), (0,))) for j in range(WB)], axis=0) + lepe
    res = mm(o, wcat[3 * c_dim:4 * c_dim], ((1,), (1,))) + misc[3:4]
    out_ref[...] = res.reshape(WB, p_dim, c_dim)


def kernel(x, Wq, bq, Wk, bk, Wv, bv, Wp, bp, lepe_w, lepe_b):
    b, c, h, w = x.shape
    gh, gw = h // GS, w // GS
    n = b * gh * gw
    p = GS * GS

    xw = (jnp.transpose(x, (0, 2, 3, 1))
          .reshape(b * gh, GS, gw, GS, c)
          .transpose(0, 2, 1, 3, 4)
          .reshape(n, p, c))

    pool_blk = min(128, n)
    d = pl.pallas_call(
        _pool_body,
        grid=(n // pool_blk,),
        in_specs=[pl.BlockSpec((pool_blk, p, c), lambda i: (i, 0, 0))],
        out_specs=pl.BlockSpec((pool_blk, c), lambda i: (i, 0)),
        out_shape=jax.ShapeDtypeStruct((n, c), jnp.float32),
    )(xw)

    r = min(256, n)
    sval, sidx = pl.pallas_call(
        _topk_body,
        grid=(n // r,),
        in_specs=[pl.BlockSpec((r, c), lambda i: (i, 0)),
                  pl.BlockSpec((n, c), lambda i: (0, 0))],
        out_specs=[pl.BlockSpec((r, 8), lambda i: (i, 0)),
                   pl.BlockSpec((r, 8), lambda i: (i, 0))],
        out_shape=[jax.ShapeDtypeStruct((n, 8), jnp.float32),
                   jax.ShapeDtypeStruct((n, 8), jnp.int32)],
    )(d, d)

    sidx_flat = sidx[:, :TOPK].reshape(-1)
    sval3 = sval.reshape(n, 1, 8)
    lt = lepe_w.reshape(c, 9).transpose(1, 0)
    wcat = jnp.concatenate([Wq, Wk, Wv, Wp], axis=0)
    misc = jnp.concatenate(
        [bq.reshape(1, c), bk.reshape(1, c), bv.reshape(1, c),
         bp.reshape(1, c), lt, lepe_b.reshape(1, c),
         jnp.zeros((2, c), jnp.float32)], axis=0)

    nspec = lambda t: pl.BlockSpec(
        (1, p, c), lambda i, si, t=t: (si[TOPK * WB * i + t], 0, 0))
    grid_spec = pltpu.PrefetchScalarGridSpec(
        num_scalar_prefetch=1,
        grid=(n // WB,),
        in_specs=[
            pl.BlockSpec((WB, p, c), lambda i, si: (i, 0, 0)),
            *[nspec(t) for t in range(TOPK * WB)],
            pl.BlockSpec((WB, 1, 8), lambda i, si: (i, 0, 0)),
            pl.BlockSpec((4 * c, c), lambda i, si: (0, 0)),
            pl.BlockSpec((16, c), lambda i, si: (0, 0)),
        ],
        out_specs=pl.BlockSpec((WB, p, c), lambda i, si: (i, 0, 0)),
    )
    out = pl.pallas_call(
        _attn_body,
        grid_spec=grid_spec,
        out_shape=jax.ShapeDtypeStruct((n, p, c), jnp.float32),
    )(sidx_flat, xw, *([xw] * (TOPK * WB)), sval3, wcat, misc)

    out = (out.reshape(b, gh, gw, GS, GS, c)
           .transpose(0, 5, 1, 3, 2, 4)
           .reshape(b, c, h, w))
    return out


# speedup vs baseline: 1.7906x; 1.0531x over previous
import jax
import jax.numpy as jnp
from jax import lax
from jax.experimental import pallas as pl
from jax.experimental.pallas import tpu as pltpu

GS = 8
TOPK = 3


def _pool_body(xw_ref, d_ref):
    d_ref[...] = jnp.max(xw_ref[...], axis=1)


def _topk_body(d_rows_ref, d_all_ref, val_ref, idx_ref):
    s = lax.dot_general(d_rows_ref[...], d_all_ref[...],
                        (((1,), (1,)), ((), ())),
                        preferred_element_type=jnp.float32)
    iota = lax.broadcasted_iota(jnp.int32, s.shape, 1)
    vals, idxs = [], []
    for _ in range(TOPK):
        v = jnp.max(s, axis=1, keepdims=True)
        i = jnp.min(jnp.where(s >= v, iota, jnp.int32(2 ** 30)),
                    axis=1, keepdims=True)
        vals.append(v)
        idxs.append(i)
        s = jnp.where(iota == i, -jnp.inf, s)
    r = s.shape[0]
    val_ref[...] = jnp.concatenate(
        vals + [jnp.full((r, 8 - TOPK), -1e30, jnp.float32)], axis=1)
    idx_ref[...] = jnp.concatenate(
        idxs + [jnp.zeros((r, 8 - TOPK), jnp.int32)], axis=1)


WB = 32


def _attn_body(*refs):
    (sidx_ref, xs_ref), nrefs = refs[:2], refs[2:2 + TOPK * WB]
    sv_ref, wcat_ref, misc_ref, out_ref = refs[2 + TOPK * WB:]
    p_dim = GS * GS
    c_dim = xs_ref.shape[-1]
    rows = WB * p_dim

    xs = xs_ref[...].reshape(rows, c_dim)

    sv = sv_ref[...].reshape(WB, 8)
    lane = lax.broadcasted_iota(jnp.int32, sv.shape, 1)
    m3 = jnp.max(jnp.where(lane < TOPK, sv, -jnp.inf), axis=1, keepdims=True)
    e = jnp.where(lane < TOPK, jnp.exp(sv - m3), 0.0)
    w = e / jnp.sum(e, axis=1, keepdims=True)

    ctx = jnp.concatenate([
        sum(w[j:j + 1, k:k + 1] * nrefs[TOPK * j + k][0] for k in range(TOPK))
        for j in range(WB)], axis=0)

    def mm(a, b, dims):
        return lax.dot_general(a, b, (dims, ((), ())),
                               preferred_element_type=jnp.float32)

    wcat = wcat_ref[...]
    misc = misc_ref[...]
    q = mm(xs, wcat[0:c_dim], ((1,), (1,))) + misc[0:1]
    kk = mm(ctx, wcat[c_dim:2 * c_dim], ((1,), (1,))) + misc[1:2]
    vv = mm(ctx, wcat[2 * c_dim:3 * c_dim], ((1,), (1,))) + misc[2:3]

    pvec = lax.broadcasted_iota(jnp.int32, (rows, 1), 0)
    yy = (pvec % p_dim) // GS
    xx = pvec % GS
    acc = jnp.zeros((rows, c_dim), jnp.float32)
    for ky in range(3):
        for kx in range(3):
            dy, dx = ky - 1, kx - 1
            s = GS * dy + dx
            sh = xs if s == 0 else jnp.roll(xs, -s, axis=0)
            valid = ((yy + dy >= 0) & (yy + dy < GS)
                     & (xx + dx >= 0) & (xx + dx < GS))
            t = 3 * ky + kx
            acc = acc + jnp.where(valid, sh, 0.0) * misc[4 + t:5 + t, :]
    lepe = acc + misc[13:14]

    scale = c_dim ** -0.5
    attn = jnp.concatenate(
        [mm(q[j * p_dim:(j + 1) * p_dim], kk[j * p_dim:(j + 1) * p_dim],
            ((1,), (1,))) for j in range(WB)], axis=0) * scale
    mrow = jnp.max(attn, axis=1, keepdims=True)
    pexp = jnp.exp(attn - mrow)
    pnorm = pexp / jnp.sum(pexp, axis=1, keepdims=True)
    o = jnp.concatenate(
        [mm(pnorm[j * p_dim:(j + 1) * p_dim], vv[j * p_dim:(j + 1) * p_dim],
            ((1,), (0,))) for j in range(WB)], axis=0) + lepe
    res = mm(o, wcat[3 * c_dim:4 * c_dim], ((1,), (1,))) + misc[3:4]
    out_ref[...] = res.reshape(WB, p_dim, c_dim)


def kernel(x, Wq, bq, Wk, bk, Wv, bv, Wp, bp, lepe_w, lepe_b):
    b, c, h, w = x.shape
    gh, gw = h // GS, w // GS
    n = b * gh * gw
    p = GS * GS

    xw = (jnp.transpose(x, (0, 2, 3, 1))
          .reshape(b * gh, GS, gw, GS, c)
          .transpose(0, 2, 1, 3, 4)
          .reshape(n, p, c))

    pool_blk = min(128, n)
    d = pl.pallas_call(
        _pool_body,
        grid=(n // pool_blk,),
        in_specs=[pl.BlockSpec((pool_blk, p, c), lambda i: (i, 0, 0))],
        out_specs=pl.BlockSpec((pool_blk, c), lambda i: (i, 0)),
        out_shape=jax.ShapeDtypeStruct((n, c), jnp.float32),
    )(xw)

    r = min(256, n)
    sval, sidx = pl.pallas_call(
        _topk_body,
        grid=(n // r,),
        in_specs=[pl.BlockSpec((r, c), lambda i: (i, 0)),
                  pl.BlockSpec((n, c), lambda i: (0, 0))],
        out_specs=[pl.BlockSpec((r, 8), lambda i: (i, 0)),
                   pl.BlockSpec((r, 8), lambda i: (i, 0))],
        out_shape=[jax.ShapeDtypeStruct((n, 8), jnp.float32),
                   jax.ShapeDtypeStruct((n, 8), jnp.int32)],
    )(d, d)

    sidx_flat = sidx[:, :TOPK].reshape(-1)
    sval3 = sval.reshape(n, 1, 8)
    lt = lepe_w.reshape(c, 9).transpose(1, 0)
    wcat = jnp.concatenate([Wq, Wk, Wv, Wp], axis=0)
    misc = jnp.concatenate(
        [bq.reshape(1, c), bk.reshape(1, c), bv.reshape(1, c),
         bp.reshape(1, c), lt, lepe_b.reshape(1, c),
         jnp.zeros((2, c), jnp.float32)], axis=0)

    nspec = lambda t: pl.BlockSpec(
        (1, p, c), lambda i, si, t=t: (si[TOPK * WB * i + t], 0, 0))
    grid_spec = pltpu.PrefetchScalarGridSpec(
        num_scalar_prefetch=1,
        grid=(n // WB,),
        in_specs=[
            pl.BlockSpec((WB, p, c), lambda i, si: (i, 0, 0)),
            *[nspec(t) for t in range(TOPK * WB)],
            pl.BlockSpec((WB, 1, 8), lambda i, si: (i, 0, 0)),
            pl.BlockSpec((4 * c, c), lambda i, si: (0, 0)),
            pl.BlockSpec((16, c), lambda i, si: (0, 0)),
        ],
        out_specs=pl.BlockSpec((WB, p, c), lambda i, si: (i, 0, 0)),
    )
    out = pl.pallas_call(
        _attn_body,
        grid_spec=grid_spec,
        out_shape=jax.ShapeDtypeStruct((n, p, c), jnp.float32),
    )(sidx_flat, xw, *([xw] * (TOPK * WB)), sval3, wcat, misc)

    out = (out.reshape(b, gh, gw, GS, GS, c)
           .transpose(0, 5, 1, 3, 2, 4)
           .reshape(b, c, h, w))
    return out


# speedup vs baseline: 1.8219x; 1.0174x over previous
import jax
import jax.numpy as jnp
from jax import lax
from jax.experimental import pallas as pl
from jax.experimental.pallas import tpu as pltpu

GS = 8
TOPK = 3


def _pool_body(xw_ref, d_ref):
    d_ref[...] = jnp.max(xw_ref[...], axis=1)


def _topk_body(d_rows_ref, d_all_ref, val_ref, idx_ref):
    s = lax.dot_general(d_rows_ref[...], d_all_ref[...],
                        (((1,), (1,)), ((), ())),
                        preferred_element_type=jnp.float32)
    iota = lax.broadcasted_iota(jnp.int32, s.shape, 1)
    vals, idxs = [], []
    for _ in range(TOPK):
        v = jnp.max(s, axis=1, keepdims=True)
        i = jnp.min(jnp.where(s >= v, iota, jnp.int32(2 ** 30)),
                    axis=1, keepdims=True)
        vals.append(v)
        idxs.append(i)
        s = jnp.where(iota == i, -jnp.inf, s)
    r = s.shape[0]
    val_ref[...] = jnp.concatenate(
        vals + [jnp.full((r, 8 - TOPK), -1e30, jnp.float32)], axis=1)
    idx_ref[...] = jnp.concatenate(
        idxs + [jnp.zeros((r, 8 - TOPK), jnp.int32)], axis=1)


WB = 64


def _attn_body(*refs):
    (sidx_ref, xs_ref), nrefs = refs[:2], refs[2:2 + TOPK * WB]
    sv_ref, wcat_ref, misc_ref, out_ref = refs[2 + TOPK * WB:]
    p_dim = GS * GS
    c_dim = xs_ref.shape[-1]
    rows = WB * p_dim

    xs = xs_ref[...].reshape(rows, c_dim)

    sv = sv_ref[...].reshape(WB, 8)
    lane = lax.broadcasted_iota(jnp.int32, sv.shape, 1)
    m3 = jnp.max(jnp.where(lane < TOPK, sv, -jnp.inf), axis=1, keepdims=True)
    e = jnp.where(lane < TOPK, jnp.exp(sv - m3), 0.0)
    w = e / jnp.sum(e, axis=1, keepdims=True)

    ctx = jnp.concatenate([
        sum(w[j:j + 1, k:k + 1] * nrefs[TOPK * j + k][0] for k in range(TOPK))
        for j in range(WB)], axis=0)

    def mm(a, b, dims):
        return lax.dot_general(a, b, (dims, ((), ())),
                               preferred_element_type=jnp.float32)

    wcat = wcat_ref[...]
    misc = misc_ref[...]
    q = mm(xs, wcat[0:c_dim], ((1,), (1,))) + misc[0:1]
    kk = mm(ctx, wcat[c_dim:2 * c_dim], ((1,), (1,))) + misc[1:2]
    vv = mm(ctx, wcat[2 * c_dim:3 * c_dim], ((1,), (1,))) + misc[2:3]

    pvec = lax.broadcasted_iota(jnp.int32, (rows, 1), 0)
    yy = (pvec % p_dim) // GS
    xx = pvec % GS
    acc = jnp.zeros((rows, c_dim), jnp.float32)
    for ky in range(3):
        for kx in range(3):
            dy, dx = ky - 1, kx - 1
            s = GS * dy + dx
            sh = xs if s == 0 else jnp.roll(xs, -s, axis=0)
            valid = ((yy + dy >= 0) & (yy + dy < GS)
                     & (xx + dx >= 0) & (xx + dx < GS))
            t = 3 * ky + kx
            acc = acc + jnp.where(valid, sh, 0.0) * misc[4 + t:5 + t, :]
    lepe = acc + misc[13:14]

    scale = c_dim ** -0.5
    attn = jnp.concatenate(
        [mm(q[j * p_dim:(j + 1) * p_dim], kk[j * p_dim:(j + 1) * p_dim],
            ((1,), (1,))) for j in range(WB)], axis=0) * scale
    mrow = jnp.max(attn, axis=1, keepdims=True)
    pexp = jnp.exp(attn - mrow)
    pnorm = pexp / jnp.sum(pexp, axis=1, keepdims=True)
    o = jnp.concatenate(
        [mm(pnorm[j * p_dim:(j + 1) * p_dim], vv[j * p_dim:(j + 1) * p_dim],
            ((1,), (0,))) for j in range(WB)], axis=0) + lepe
    res = mm(o, wcat[3 * c_dim:4 * c_dim], ((1,), (1,))) + misc[3:4]
    out_ref[...] = res.reshape(WB, p_dim, c_dim)


def kernel(x, Wq, bq, Wk, bk, Wv, bv, Wp, bp, lepe_w, lepe_b):
    b, c, h, w = x.shape
    gh, gw = h // GS, w // GS
    n = b * gh * gw
    p = GS * GS

    xw = (jnp.transpose(x, (0, 2, 3, 1))
          .reshape(b * gh, GS, gw, GS, c)
          .transpose(0, 2, 1, 3, 4)
          .reshape(n, p, c))

    pool_blk = min(128, n)
    d = pl.pallas_call(
        _pool_body,
        grid=(n // pool_blk,),
        in_specs=[pl.BlockSpec((pool_blk, p, c), lambda i: (i, 0, 0))],
        out_specs=pl.BlockSpec((pool_blk, c), lambda i: (i, 0)),
        out_shape=jax.ShapeDtypeStruct((n, c), jnp.float32),
    )(xw)

    r = min(256, n)
    sval, sidx = pl.pallas_call(
        _topk_body,
        grid=(n // r,),
        in_specs=[pl.BlockSpec((r, c), lambda i: (i, 0)),
                  pl.BlockSpec((n, c), lambda i: (0, 0))],
        out_specs=[pl.BlockSpec((r, 8), lambda i: (i, 0)),
                   pl.BlockSpec((r, 8), lambda i: (i, 0))],
        out_shape=[jax.ShapeDtypeStruct((n, 8), jnp.float32),
                   jax.ShapeDtypeStruct((n, 8), jnp.int32)],
    )(d, d)

    sidx_flat = sidx[:, :TOPK].reshape(-1)
    sval3 = sval.reshape(n, 1, 8)
    lt = lepe_w.reshape(c, 9).transpose(1, 0)
    wcat = jnp.concatenate([Wq, Wk, Wv, Wp], axis=0)
    misc = jnp.concatenate(
        [bq.reshape(1, c), bk.reshape(1, c), bv.reshape(1, c),
         bp.reshape(1, c), lt, lepe_b.reshape(1, c),
         jnp.zeros((2, c), jnp.float32)], axis=0)

    nspec = lambda t: pl.BlockSpec(
        (1, p, c), lambda i, si, t=t: (si[TOPK * WB * i + t], 0, 0))
    grid_spec = pltpu.PrefetchScalarGridSpec(
        num_scalar_prefetch=1,
        grid=(n // WB,),
        in_specs=[
            pl.BlockSpec((WB, p, c), lambda i, si: (i, 0, 0)),
            *[nspec(t) for t in range(TOPK * WB)],
            pl.BlockSpec((WB, 1, 8), lambda i, si: (i, 0, 0)),
            pl.BlockSpec((4 * c, c), lambda i, si: (0, 0)),
            pl.BlockSpec((16, c), lambda i, si: (0, 0)),
        ],
        out_specs=pl.BlockSpec((WB, p, c), lambda i, si: (i, 0, 0)),
    )
    out = pl.pallas_call(
        _attn_body,
        grid_spec=grid_spec,
        out_shape=jax.ShapeDtypeStruct((n, p, c), jnp.float32),
    )(sidx_flat, xw, *([xw] * (TOPK * WB)), sval3, wcat, misc)

    out = (out.reshape(b, gh, gw, GS, GS, c)
           .transpose(0, 5, 1, 3, 2, 4)
           .reshape(b, c, h, w))
    return out
